# baseline (speedup 1.0000x reference)
import functools

import jax
import jax.numpy as jnp
from jax import lax
from jax.experimental import pallas as pl
from jax.experimental.pallas import tpu as pltpu
from jax.experimental.pallas import tpu_sc as plsc

N = 2048
D = 768
H = 2048
E = 8
EPS = 1e-06

B = 256
NB = N // B
NWORK = NB + E - 1

NC = 2
NS = 16
L = 16
NW = NC * NS
CHUNK = N // NW
NV = N // L
CV = CHUNK // L


def _sc_wid():
    return lax.axis_index("s") * NC + lax.axis_index("c")


def _routing_body(at_hbm, x_hbm,
                  pos_hbm, xs_hbm, ew_hbm, nb_hbm, val_hbm, st_hbm, en_hbm,
                  at_v, atc_v, xrows_v, pos_v, hist_pre, hist_tot,
                  base_v, off_v, lo_v, ws_v, ew_v, nb_v, out16_v, sem):
    wid = _sc_wid()
    base = wid * CHUNK
    pltpu.sync_copy(at_hbm, at_v)
    pltpu.sync_copy(at_hbm.at[pl.ds(base, CHUNK)], atc_v)
    zero = jnp.zeros((L,), jnp.int32)
    ones = jnp.full((L,), 1, jnp.int32)
    true_m = jnp.full((L,), True)
    hist_pre[...] = zero
    hist_tot[...] = zero
    myv0 = wid * CV
    for v in range(NV):
        a = at_v[pl.ds(v * L, L)]
        plsc.addupdate_scatter(hist_tot, [a], ones, mask=true_m)
        m = jnp.full((L,), v, jnp.int32) < myv0
        plsc.addupdate_scatter(hist_pre, [a], ones, mask=m)
    tot = hist_tot[...]
    excl = plsc.cumsum(tot) - tot
    base_v[...] = excl + hist_pre[...]
    for v in range(CV):
        a = atc_v[pl.ds(v * L, L)]
        bg = plsc.load_gather(base_v, [a])
        rank = jnp.zeros((L,), jnp.int32)
        for e in range(E):
            m = a == e
            c = plsc.cumsum(m.astype(jnp.int32)) - 1
            rank = jnp.where(m, c, rank)
        pos_v[pl.ds(v * L, L)] = bg + rank
        plsc.addupdate_scatter(base_v, [a], ones, mask=true_m)
    pltpu.sync_copy(pos_v, pos_hbm.at[pl.ds(base, CHUNK)])
    pltpu.sync_copy(x_hbm.at[pl.ds(base, CHUNK)], xrows_v)
    pltpu.async_copy(xrows_v, xs_hbm.at[pos_v], sem).wait()

    @pl.when(wid == 0)
    def _():
        lane = lax.iota(jnp.int32, L)
        off_v[...] = excl
        offp1 = plsc.load_gather(off_v, [jnp.minimum(lane + 1, L - 1)])
        lo = excl // B
        nonempty = (offp1 > excl) & (lane < E)
        hi = jnp.where(nonempty, (jnp.maximum(offp1, 1) - 1) // B, lo)
        nblk = jnp.where(nonempty, hi - lo + 1, 0)
        winc = plsc.cumsum(nblk)
        wstart = winc - nblk
        lo_v[...] = lo
        ws_v[...] = wstart
        cnt_v = out16_v
        cnt_v[...] = zero
        plsc.addupdate_scatter(cnt_v, [wstart], ones, mask=lane < E)
        acc = plsc.cumsum(cnt_v[...])
        ew_raw = acc - 1
        lo_g = plsc.load_gather(lo_v, [ew_raw])
        ws_g = plsc.load_gather(ws_v, [ew_raw])
        nb_raw = lo_g + lane - ws_g
        ew_v[...] = ew_raw
        nb_v[...] = nb_raw
        nwork = jnp.max(winc)
        valid = lane < nwork
        last_v = lane * 0 + jnp.maximum(nwork - 1, 0)
        ew_last = plsc.load_gather(ew_v, [last_v])
        nb_last = plsc.load_gather(nb_v, [last_v])
        ew = jnp.where(valid, ew_raw, ew_last)
        nb = jnp.where(valid, nb_raw, nb_last)
        st = plsc.load_gather(off_v, [ew])
        en = plsc.load_gather(off_v, [ew + 1])
        out16_v[...] = ew
        pltpu.sync_copy(out16_v, ew_hbm)
        out16_v[...] = nb
        pltpu.sync_copy(out16_v, nb_hbm)
        out16_v[...] = valid.astype(jnp.int32)
        pltpu.sync_copy(out16_v, val_hbm)
        out16_v[...] = st
        pltpu.sync_copy(out16_v, st_hbm)
        out16_v[...] = en
        pltpu.sync_copy(out16_v, en_hbm)


def _unsort_body(zs_hbm, pos_hbm, out_hbm, pos_v, rows_v, sem):
    wid = _sc_wid()
    base = wid * CHUNK
    pltpu.sync_copy(pos_hbm.at[pl.ds(base, CHUNK)], pos_v)
    pltpu.async_copy(zs_hbm.at[pos_v], rows_v, sem).wait()
    pltpu.sync_copy(rows_v, out_hbm.at[pl.ds(base, CHUNK)])


def _sc_mesh():
    return plsc.VectorSubcoreMesh(core_axis_name="c", subcore_axis_name="s",
                                  num_cores=NC, num_subcores=NS)

_I16 = jax.ShapeDtypeStruct((L,), jnp.int32)


@jax.jit
def _routing_sc(at, x):
    return pl.kernel(
        _routing_body,
        out_type=(
            jax.ShapeDtypeStruct((N,), jnp.int32),
            jax.ShapeDtypeStruct((N, D), jnp.float32),
            _I16, _I16, _I16, _I16, _I16,
        ),
        mesh=_sc_mesh(),
        compiler_params=pltpu.CompilerParams(needs_layout_passes=False),
        scratch_types=[
            pltpu.VMEM((N,), jnp.int32),
            pltpu.VMEM((CHUNK,), jnp.int32),
            pltpu.VMEM((CHUNK, D), jnp.float32),
            pltpu.VMEM((CHUNK,), jnp.int32),
        ] + [pltpu.VMEM((L,), jnp.int32)] * 9 + [
            pltpu.SemaphoreType.DMA,
        ],
    )(at, x)


@jax.jit
def _unsort_sc(zs, pos):
    return pl.kernel(
        _unsort_body,
        out_type=jax.ShapeDtypeStruct((N, D), jnp.float32),
        mesh=_sc_mesh(),
        compiler_params=pltpu.CompilerParams(needs_layout_passes=False),
        scratch_types=[
            pltpu.VMEM((CHUNK,), jnp.int32),
            pltpu.VMEM((CHUNK, D), jnp.float32),
            pltpu.SemaphoreType.DMA,
        ],
    )(zs, pos)


def _mlp_body(ew_ref, nb_ref, val_ref, st_ref, en_ref,
              x_ref, W1p_ref, W1g_ref, b1_ref, W2a_ref, W2b_ref, b2_ref,
              g_ref, out_ref):
    w = pl.program_id(0)

    @pl.when(val_ref[w] == 1)
    def _():
        blk_lo = nb_ref[w] * B
        start = st_ref[w]
        end = en_ref[w]
        x = x_ref[...]
        proj = jax.lax.dot_general(
            x, W1p_ref[0], (((1,), (1,)), ((), ())),
            preferred_element_type=jnp.float32) + b1_ref[0, :, :H]
        gate = jax.lax.dot_general(
            x, W1g_ref[0], (((1,), (1,)), ((), ())),
            preferred_element_type=jnp.float32) + b1_ref[0, :, H:]
        h = proj * (gate * jax.lax.logistic(gate))
        ya = jax.lax.dot_general(
            h, W2a_ref[0], (((1,), (1,)), ((), ())),
            preferred_element_type=jnp.float32)
        yb = jax.lax.dot_general(
            h, W2b_ref[0], (((1,), (1,)), ((), ())),
            preferred_element_type=jnp.float32)
        y = jnp.concatenate([ya, yb], axis=1) + b2_ref[0]
        z = x + y
        ms = jnp.mean(z * z, axis=-1, keepdims=True)
        z = z * jax.lax.rsqrt(ms + EPS) * g_ref[0]
        gid = blk_lo + jax.lax.broadcasted_iota(jnp.int32, (B, 1), 0)
        mask = (gid >= start) & (gid < end)
        out_ref[...] = jnp.where(mask, z, out_ref[...])


@functools.partial(jax.jit, static_argnames=("interpret",))
def _mlp_sorted(ew, nb, val, st, en, x_sorted, W1, b1, W2, b2, g,
                interpret=False):
    grid_spec = pltpu.PrefetchScalarGridSpec(
        num_scalar_prefetch=5,
        grid=(NWORK,),
        in_specs=[
            pl.BlockSpec((B, D), lambda w, ew, nb, val, st, en: (nb[w], 0)),
            pl.BlockSpec((1, H, D),
                         lambda w, ew, nb, val, st, en: (ew[w], 0, 0)),
            pl.BlockSpec((1, H, D),
                         lambda w, ew, nb, val, st, en: (ew[w], 1, 0)),
            pl.BlockSpec((1, 1, 2 * H),
                         lambda w, ew, nb, val, st, en: (ew[w], 0, 0)),
            pl.BlockSpec((1, D // 2, H),
                         lambda w, ew, nb, val, st, en: (ew[w], 0, 0)),
            pl.BlockSpec((1, D // 2, H),
                         lambda w, ew, nb, val, st, en: (ew[w], 1, 0)),
            pl.BlockSpec((1, 1, D),
                         lambda w, ew, nb, val, st, en: (ew[w], 0, 0)),
            pl.BlockSpec((1, D), lambda w, ew, nb, val, st, en: (0, 0)),
        ],
        out_specs=pl.BlockSpec((B, D), lambda w, ew, nb, val, st, en: (nb[w], 0)),
    )
    return pl.pallas_call(
        _mlp_body,
        grid_spec=grid_spec,
        out_shape=jax.ShapeDtypeStruct((N, D), jnp.float32),
        compiler_params=pltpu.CompilerParams(
            dimension_semantics=("arbitrary",)),
        interpret=interpret,
    )(ew, nb, val, st, en, x_sorted,
      W1, W1, b1.reshape(E, 1, 2 * H), W2, W2, b2.reshape(E, 1, D),
      g.reshape(1, D))


def kernel(x, action_type, W1, b1, W2, b2, g, interpret=False):
    at = action_type.astype(jnp.int32)
    pos, x_sorted, ew, nb, val, st, en = _routing_sc(at, x)
    z_sorted = _mlp_sorted(ew, nb, val, st, en, x_sorted, W1, b1, W2, b2, g,
                           interpret=interpret)
    return _unsort_sc(z_sorted, pos)


# speedup vs baseline: 1.0290x; 1.0290x over previous
import functools

import jax
import jax.numpy as jnp
from jax import lax
from jax.experimental import pallas as pl
from jax.experimental.pallas import tpu as pltpu
from jax.experimental.pallas import tpu_sc as plsc

N = 2048
D = 768
H = 2048
E = 8
EPS = 1e-06

B = 256
NB = N // B
NWORK = NB + E - 1

NC = 2
NS = 16
L = 16
NW = NC * NS
CHUNK = N // NW
NV = N // L
CV = CHUNK // L


def _sc_wid():
    return lax.axis_index("s") * NC + lax.axis_index("c")


def _routing_body(at_hbm, x_hbm,
                  pos_hbm, xs_hbm, ew_hbm, nb_hbm, val_hbm, st_hbm, en_hbm,
                  at_v, atc_v, xrows_v, pos_v, hist_pre, hist_tot,
                  base_v, off_v, lo_v, ws_v, ew_v, nb_v, out16_v, sem):
    wid = _sc_wid()
    base = wid * CHUNK
    pltpu.sync_copy(at_hbm, at_v)
    pltpu.sync_copy(at_hbm.at[pl.ds(base, CHUNK)], atc_v)
    zero = jnp.zeros((L,), jnp.int32)
    ones = jnp.full((L,), 1, jnp.int32)
    true_m = jnp.full((L,), True)
    hist_pre[...] = zero
    hist_tot[...] = zero
    myv0 = wid * CV
    for v in range(NV):
        a = at_v[pl.ds(v * L, L)]
        plsc.addupdate_scatter(hist_tot, [a], ones, mask=true_m)
        m = jnp.full((L,), v, jnp.int32) < myv0
        plsc.addupdate_scatter(hist_pre, [a], ones, mask=m)
    tot = hist_tot[...]
    excl = plsc.cumsum(tot) - tot
    base_v[...] = excl + hist_pre[...]
    for v in range(CV):
        a = atc_v[pl.ds(v * L, L)]
        bg = plsc.load_gather(base_v, [a])
        rank = jnp.zeros((L,), jnp.int32)
        for e in range(E):
            m = a == e
            c = plsc.cumsum(m.astype(jnp.int32)) - 1
            rank = jnp.where(m, c, rank)
        pos_v[pl.ds(v * L, L)] = bg + rank
        plsc.addupdate_scatter(base_v, [a], ones, mask=true_m)
    pltpu.sync_copy(pos_v, pos_hbm.at[pl.ds(base, CHUNK)])
    pltpu.sync_copy(x_hbm.at[pl.ds(base, CHUNK)], xrows_v)
    pltpu.async_copy(xrows_v, xs_hbm.at[pos_v], sem).wait()

    @pl.when(wid == 0)
    def _():
        lane = lax.iota(jnp.int32, L)
        off_v[...] = excl
        offp1 = plsc.load_gather(off_v, [jnp.minimum(lane + 1, L - 1)])
        lo = excl // B
        nonempty = (offp1 > excl) & (lane < E)
        hi = jnp.where(nonempty, (jnp.maximum(offp1, 1) - 1) // B, lo)
        nblk = jnp.where(nonempty, hi - lo + 1, 0)
        winc = plsc.cumsum(nblk)
        wstart = winc - nblk
        lo_v[...] = lo
        ws_v[...] = wstart
        cnt_v = out16_v
        cnt_v[...] = zero
        plsc.addupdate_scatter(cnt_v, [wstart], ones, mask=lane < E)
        acc = plsc.cumsum(cnt_v[...])
        ew_raw = acc - 1
        lo_g = plsc.load_gather(lo_v, [ew_raw])
        ws_g = plsc.load_gather(ws_v, [ew_raw])
        nb_raw = lo_g + lane - ws_g
        ew_v[...] = ew_raw
        nb_v[...] = nb_raw
        nwork = jnp.max(winc)
        valid = lane < nwork
        last_v = lane * 0 + jnp.maximum(nwork - 1, 0)
        ew_last = plsc.load_gather(ew_v, [last_v])
        nb_last = plsc.load_gather(nb_v, [last_v])
        ew = jnp.where(valid, ew_raw, ew_last)
        nb = jnp.where(valid, nb_raw, nb_last)
        st = plsc.load_gather(off_v, [ew])
        en = plsc.load_gather(off_v, [ew + 1])
        out16_v[...] = ew
        pltpu.sync_copy(out16_v, ew_hbm)
        out16_v[...] = nb
        pltpu.sync_copy(out16_v, nb_hbm)
        out16_v[...] = valid.astype(jnp.int32)
        pltpu.sync_copy(out16_v, val_hbm)
        out16_v[...] = st
        pltpu.sync_copy(out16_v, st_hbm)
        out16_v[...] = en
        pltpu.sync_copy(out16_v, en_hbm)


def _unsort_body(zs_hbm, pos_hbm, out_hbm, pos_v, rows_v, sem):
    wid = _sc_wid()
    base = wid * CHUNK
    pltpu.sync_copy(pos_hbm.at[pl.ds(base, CHUNK)], pos_v)
    pltpu.async_copy(zs_hbm.at[pos_v], rows_v, sem).wait()
    pltpu.sync_copy(rows_v, out_hbm.at[pl.ds(base, CHUNK)])


def _sc_mesh():
    return plsc.VectorSubcoreMesh(core_axis_name="c", subcore_axis_name="s",
                                  num_cores=NC, num_subcores=NS)

_I16 = jax.ShapeDtypeStruct((L,), jnp.int32)


@jax.jit
def _routing_sc(at, x):
    return pl.kernel(
        _routing_body,
        out_type=(
            jax.ShapeDtypeStruct((N,), jnp.int32),
            jax.ShapeDtypeStruct((N, D), jnp.float32),
            _I16, _I16, _I16, _I16, _I16,
        ),
        mesh=_sc_mesh(),
        compiler_params=pltpu.CompilerParams(needs_layout_passes=False),
        scratch_types=[
            pltpu.VMEM((N,), jnp.int32),
            pltpu.VMEM((CHUNK,), jnp.int32),
            pltpu.VMEM((CHUNK, D), jnp.float32),
            pltpu.VMEM((CHUNK,), jnp.int32),
        ] + [pltpu.VMEM((L,), jnp.int32)] * 9 + [
            pltpu.SemaphoreType.DMA,
        ],
    )(at, x)


@jax.jit
def _unsort_sc(zs, pos):
    return pl.kernel(
        _unsort_body,
        out_type=jax.ShapeDtypeStruct((N, D), jnp.float32),
        mesh=_sc_mesh(),
        compiler_params=pltpu.CompilerParams(needs_layout_passes=False),
        scratch_types=[
            pltpu.VMEM((CHUNK,), jnp.int32),
            pltpu.VMEM((CHUNK, D), jnp.float32),
            pltpu.SemaphoreType.DMA,
        ],
    )(zs, pos)


def _mlp_body(ew_ref, nb_ref, val_ref, st_ref, en_ref,
              x_ref, W1_ref, b1_ref, W2_ref, b2_ref, g_ref, out_ref):
    w = pl.program_id(0)

    @pl.when(val_ref[w] == 1)
    def _():
        blk_lo = nb_ref[w] * B
        start = st_ref[w]
        end = en_ref[w]
        x = x_ref[...]
        p = jax.lax.dot_general(
            x, W1_ref[0], (((1,), (1,)), ((), ())),
            preferred_element_type=jnp.float32)
        p = p + b1_ref[0]
        proj = p[:, :H]
        gate = p[:, H:]
        h = proj * (gate * jax.lax.logistic(gate))
        y = jax.lax.dot_general(
            h, W2_ref[0], (((1,), (1,)), ((), ())),
            preferred_element_type=jnp.float32)
        y = y + b2_ref[0]
        z = x + y
        ms = jnp.mean(z * z, axis=-1, keepdims=True)
        z = z * jax.lax.rsqrt(ms + EPS) * g_ref[0]
        gid = blk_lo + jax.lax.broadcasted_iota(jnp.int32, (B, 1), 0)
        mask = (gid >= start) & (gid < end)
        out_ref[...] = jnp.where(mask, z, out_ref[...])


@functools.partial(jax.jit, static_argnames=("interpret",))
def _mlp_sorted(ew, nb, val, st, en, x_sorted, W1, b1, W2, b2, g,
                interpret=False):
    grid_spec = pltpu.PrefetchScalarGridSpec(
        num_scalar_prefetch=5,
        grid=(NWORK,),
        in_specs=[
            pl.BlockSpec((B, D), lambda w, ew, nb, val, st, en: (nb[w], 0)),
            pl.BlockSpec((1, 2 * H, D),
                         lambda w, ew, nb, val, st, en: (ew[w], 0, 0)),
            pl.BlockSpec((1, 1, 2 * H),
                         lambda w, ew, nb, val, st, en: (ew[w], 0, 0)),
            pl.BlockSpec((1, D, H),
                         lambda w, ew, nb, val, st, en: (ew[w], 0, 0)),
            pl.BlockSpec((1, 1, D),
                         lambda w, ew, nb, val, st, en: (ew[w], 0, 0)),
            pl.BlockSpec((1, D), lambda w, ew, nb, val, st, en: (0, 0)),
        ],
        out_specs=pl.BlockSpec((B, D), lambda w, ew, nb, val, st, en: (nb[w], 0)),
    )
    return pl.pallas_call(
        _mlp_body,
        grid_spec=grid_spec,
        out_shape=jax.ShapeDtypeStruct((N, D), jnp.float32),
        compiler_params=pltpu.CompilerParams(
            dimension_semantics=("arbitrary",)),
        interpret=interpret,
    )(ew, nb, val, st, en, x_sorted,
      W1, b1.reshape(E, 1, 2 * H), W2, b2.reshape(E, 1, D),
      g.reshape(1, D))


def kernel(x, action_type, W1, b1, W2, b2, g, interpret=False):
    at = action_type.astype(jnp.int32)
    pos, x_sorted, ew, nb, val, st, en = _routing_sc(at, x)
    z_sorted = _mlp_sorted(ew, nb, val, st, en, x_sorted, W1, b1, W2, b2, g,
                           interpret=interpret)
    return _unsort_sc(z_sorted, pos)


# speedup vs baseline: 1.0416x; 1.0122x over previous
import functools

import jax
import jax.numpy as jnp
from jax import lax
from jax.experimental import pallas as pl
from jax.experimental.pallas import tpu as pltpu
from jax.experimental.pallas import tpu_sc as plsc

N = 2048
D = 768
H = 2048
E = 8
EPS = 1e-06

B = 256
NB = N // B
NWORK = NB + E - 1

NC = 2
NS = 16
L = 16
NW = NC * NS
CHUNK = N // NW
NV = N // L
CV = CHUNK // L


def _sc_wid():
    return lax.axis_index("s") * NC + lax.axis_index("c")


def _routing_body(at_hbm, x_hbm,
                  pos_hbm, xs_hbm, ew_hbm, nb_hbm, val_hbm, st_hbm, en_hbm,
                  at_v, atc_v, xrows_v, pos_v, hist_pre, hist_tot,
                  base_v, off_v, lo_v, ws_v, ew_v, nb_v, out16_v, sem, sem2):
    wid = _sc_wid()
    base = wid * CHUNK
    cp_x = pltpu.async_copy(x_hbm.at[pl.ds(base, CHUNK)], xrows_v, sem2)
    pltpu.sync_copy(at_hbm, at_v)
    pltpu.sync_copy(at_hbm.at[pl.ds(base, CHUNK)], atc_v)
    zero = jnp.zeros((L,), jnp.int32)
    ones = jnp.full((L,), 1, jnp.int32)
    true_m = jnp.full((L,), True)
    hist_pre[...] = zero
    hist_tot[...] = zero
    myv0 = wid * CV
    for v in range(NV):
        a = at_v[pl.ds(v * L, L)]
        plsc.addupdate_scatter(hist_tot, [a], ones, mask=true_m)
        m = jnp.full((L,), v, jnp.int32) < myv0
        plsc.addupdate_scatter(hist_pre, [a], ones, mask=m)
    tot = hist_tot[...]
    excl = plsc.cumsum(tot) - tot
    base_v[...] = excl + hist_pre[...]
    for v in range(CV):
        a = atc_v[pl.ds(v * L, L)]
        bg = plsc.load_gather(base_v, [a])
        rank = jnp.zeros((L,), jnp.int32)
        for e in range(E):
            m = a == e
            c = plsc.cumsum(m.astype(jnp.int32)) - 1
            rank = jnp.where(m, c, rank)
        pos_v[pl.ds(v * L, L)] = bg + rank
        plsc.addupdate_scatter(base_v, [a], ones, mask=true_m)
    cp_pos = pltpu.async_copy(pos_v, pos_hbm.at[pl.ds(base, CHUNK)], sem)
    cp_x.wait()
    cp_xs = pltpu.async_copy(xrows_v, xs_hbm.at[pos_v], sem2)
    cp_pos.wait()
    cp_xs.wait()

    @pl.when(wid == 0)
    def _():
        lane = lax.iota(jnp.int32, L)
        off_v[...] = excl
        offp1 = plsc.load_gather(off_v, [jnp.minimum(lane + 1, L - 1)])
        lo = excl // B
        nonempty = (offp1 > excl) & (lane < E)
        hi = jnp.where(nonempty, (jnp.maximum(offp1, 1) - 1) // B, lo)
        nblk = jnp.where(nonempty, hi - lo + 1, 0)
        winc = plsc.cumsum(nblk)
        wstart = winc - nblk
        lo_v[...] = lo
        ws_v[...] = wstart
        cnt_v = out16_v
        cnt_v[...] = zero
        plsc.addupdate_scatter(cnt_v, [wstart], ones, mask=lane < E)
        acc = plsc.cumsum(cnt_v[...])
        ew_raw = acc - 1
        lo_g = plsc.load_gather(lo_v, [ew_raw])
        ws_g = plsc.load_gather(ws_v, [ew_raw])
        nb_raw = lo_g + lane - ws_g
        ew_v[...] = ew_raw
        nb_v[...] = nb_raw
        nwork = jnp.max(winc)
        valid = lane < nwork
        last_v = lane * 0 + jnp.maximum(nwork - 1, 0)
        ew_last = plsc.load_gather(ew_v, [last_v])
        nb_last = plsc.load_gather(nb_v, [last_v])
        ew = jnp.where(valid, ew_raw, ew_last)
        nb = jnp.where(valid, nb_raw, nb_last)
        st = plsc.load_gather(off_v, [ew])
        en = plsc.load_gather(off_v, [ew + 1])
        out16_v[...] = ew
        pltpu.sync_copy(out16_v, ew_hbm)
        out16_v[...] = nb
        pltpu.sync_copy(out16_v, nb_hbm)
        out16_v[...] = valid.astype(jnp.int32)
        pltpu.sync_copy(out16_v, val_hbm)
        out16_v[...] = st
        pltpu.sync_copy(out16_v, st_hbm)
        out16_v[...] = en
        pltpu.sync_copy(out16_v, en_hbm)


def _unsort_body(zs_hbm, pos_hbm, out_hbm, pos_v, rows_v,
                 sem_a, sem_b, sem_c, sem_d):
    wid = _sc_wid()
    base = wid * CHUNK
    hh = CHUNK // 2
    pltpu.sync_copy(pos_hbm.at[pl.ds(base, CHUNK)], pos_v)
    g0 = pltpu.async_copy(zs_hbm.at[pos_v.at[pl.ds(0, hh)]],
                          rows_v.at[pl.ds(0, hh)], sem_a)
    g1 = pltpu.async_copy(zs_hbm.at[pos_v.at[pl.ds(hh, hh)]],
                          rows_v.at[pl.ds(hh, hh)], sem_b)
    g0.wait()
    s0 = pltpu.async_copy(rows_v.at[pl.ds(0, hh)],
                          out_hbm.at[pl.ds(base, hh)], sem_c)
    g1.wait()
    s1 = pltpu.async_copy(rows_v.at[pl.ds(hh, hh)],
                          out_hbm.at[pl.ds(base + hh, hh)], sem_d)
    s0.wait()
    s1.wait()


def _sc_mesh():
    return plsc.VectorSubcoreMesh(core_axis_name="c", subcore_axis_name="s",
                                  num_cores=NC, num_subcores=NS)

_I16 = jax.ShapeDtypeStruct((L,), jnp.int32)


@jax.jit
def _routing_sc(at, x):
    return pl.kernel(
        _routing_body,
        out_type=(
            jax.ShapeDtypeStruct((N,), jnp.int32),
            jax.ShapeDtypeStruct((N, D), jnp.float32),
            _I16, _I16, _I16, _I16, _I16,
        ),
        mesh=_sc_mesh(),
        compiler_params=pltpu.CompilerParams(needs_layout_passes=False),
        scratch_types=[
            pltpu.VMEM((N,), jnp.int32),
            pltpu.VMEM((CHUNK,), jnp.int32),
            pltpu.VMEM((CHUNK, D), jnp.float32),
            pltpu.VMEM((CHUNK,), jnp.int32),
        ] + [pltpu.VMEM((L,), jnp.int32)] * 9 + [
            pltpu.SemaphoreType.DMA,
            pltpu.SemaphoreType.DMA,
        ],
    )(at, x)


@jax.jit
def _unsort_sc(zs, pos):
    return pl.kernel(
        _unsort_body,
        out_type=jax.ShapeDtypeStruct((N, D), jnp.float32),
        mesh=_sc_mesh(),
        compiler_params=pltpu.CompilerParams(needs_layout_passes=False),
        scratch_types=[
            pltpu.VMEM((CHUNK,), jnp.int32),
            pltpu.VMEM((CHUNK, D), jnp.float32),
            pltpu.SemaphoreType.DMA,
            pltpu.SemaphoreType.DMA,
            pltpu.SemaphoreType.DMA,
            pltpu.SemaphoreType.DMA,
        ],
    )(zs, pos)


def _mlp_body(ew_ref, nb_ref, val_ref, st_ref, en_ref,
              x_ref, W1_ref, b1_ref, W2_ref, b2_ref, g_ref, out_ref):
    w = pl.program_id(0)

    @pl.when(val_ref[w] == 1)
    def _():
        blk_lo = nb_ref[w] * B
        start = st_ref[w]
        end = en_ref[w]
        x = x_ref[...]
        p = jax.lax.dot_general(
            x, W1_ref[0], (((1,), (1,)), ((), ())),
            preferred_element_type=jnp.float32)
        p = p + b1_ref[0]
        proj = p[:, :H]
        gate = p[:, H:]
        h = proj * (gate * jax.lax.logistic(gate))
        y = jax.lax.dot_general(
            h, W2_ref[0], (((1,), (1,)), ((), ())),
            preferred_element_type=jnp.float32)
        y = y + b2_ref[0]
        z = x + y
        ms = jnp.mean(z * z, axis=-1, keepdims=True)
        z = z * jax.lax.rsqrt(ms + EPS) * g_ref[0]
        gid = blk_lo + jax.lax.broadcasted_iota(jnp.int32, (B, 1), 0)
        mask = (gid >= start) & (gid < end)
        out_ref[...] = jnp.where(mask, z, out_ref[...])


@functools.partial(jax.jit, static_argnames=("interpret",))
def _mlp_sorted(ew, nb, val, st, en, x_sorted, W1, b1, W2, b2, g,
                interpret=False):
    grid_spec = pltpu.PrefetchScalarGridSpec(
        num_scalar_prefetch=5,
        grid=(NWORK,),
        in_specs=[
            pl.BlockSpec((B, D), lambda w, ew, nb, val, st, en: (nb[w], 0)),
            pl.BlockSpec((1, 2 * H, D),
                         lambda w, ew, nb, val, st, en: (ew[w], 0, 0)),
            pl.BlockSpec((1, 1, 2 * H),
                         lambda w, ew, nb, val, st, en: (ew[w], 0, 0)),
            pl.BlockSpec((1, D, H),
                         lambda w, ew, nb, val, st, en: (ew[w], 0, 0)),
            pl.BlockSpec((1, 1, D),
                         lambda w, ew, nb, val, st, en: (ew[w], 0, 0)),
            pl.BlockSpec((1, D), lambda w, ew, nb, val, st, en: (0, 0)),
        ],
        out_specs=pl.BlockSpec((B, D), lambda w, ew, nb, val, st, en: (nb[w], 0)),
    )
    return pl.pallas_call(
        _mlp_body,
        grid_spec=grid_spec,
        out_shape=jax.ShapeDtypeStruct((N, D), jnp.float32),
        compiler_params=pltpu.CompilerParams(
            dimension_semantics=("arbitrary",)),
        interpret=interpret,
    )(ew, nb, val, st, en, x_sorted,
      W1, b1.reshape(E, 1, 2 * H), W2, b2.reshape(E, 1, D),
      g.reshape(1, D))


def kernel(x, action_type, W1, b1, W2, b2, g, interpret=False):
    at = action_type.astype(jnp.int32)
    pos, x_sorted, ew, nb, val, st, en = _routing_sc(at, x)
    z_sorted = _mlp_sorted(ew, nb, val, st, en, x_sorted, W1, b1, W2, b2, g,
                           interpret=interpret)
    return _unsort_sc(z_sorted, pos)


# speedup vs baseline: 1.0425x; 1.0009x over previous
import functools

import jax
import jax.numpy as jnp
from jax import lax
from jax.experimental import pallas as pl
from jax.experimental.pallas import tpu as pltpu
from jax.experimental.pallas import tpu_sc as plsc

N = 2048
D = 768
H = 2048
E = 8
EPS = 1e-06

B = 256
NB = N // B
NWORK = NB + E - 1

NC = 2
NS = 16
L = 16
NW = NC * NS
CHUNK = N // NW
NV = N // L
CV = CHUNK // L


def _sc_wid():
    return lax.axis_index("s") * NC + lax.axis_index("c")


def _routing_body(at_hbm, x_hbm,
                  pos_hbm, xs_hbm, ew_hbm, nb_hbm, val_hbm, st_hbm, en_hbm,
                  at_v, atc_v, xrows_v, pos_v, hist_pre, hist_tot,
                  base_v, off_v, lo_v, ws_v, ew_v, nb_v, out16_v, sem, sem2):
    wid = _sc_wid()
    base = wid * CHUNK
    cp_x = pltpu.async_copy(x_hbm.at[pl.ds(base, CHUNK)], xrows_v, sem2)
    pltpu.sync_copy(at_hbm, at_v)
    pltpu.sync_copy(at_hbm.at[pl.ds(base, CHUNK)], atc_v)
    zero = jnp.zeros((L,), jnp.int32)
    ones = jnp.full((L,), 1, jnp.int32)
    true_m = jnp.full((L,), True)
    hist_pre[...] = zero
    hist_tot[...] = zero
    myv0 = wid * CV
    for v in range(NV):
        a = at_v[pl.ds(v * L, L)]
        plsc.addupdate_scatter(hist_tot, [a], ones, mask=true_m)
        m = jnp.full((L,), v, jnp.int32) < myv0
        plsc.addupdate_scatter(hist_pre, [a], ones, mask=m)
    tot = hist_tot[...]
    excl = plsc.cumsum(tot) - tot
    base_v[...] = excl + hist_pre[...]
    for v in range(CV):
        a = atc_v[pl.ds(v * L, L)]
        bg = plsc.load_gather(base_v, [a])
        rank = jnp.zeros((L,), jnp.int32)
        for e in range(E):
            m = a == e
            c = plsc.cumsum(m.astype(jnp.int32)) - 1
            rank = jnp.where(m, c, rank)
        pos_v[pl.ds(v * L, L)] = bg + rank
        plsc.addupdate_scatter(base_v, [a], ones, mask=true_m)
    cp_pos = pltpu.async_copy(pos_v, pos_hbm.at[pl.ds(base, CHUNK)], sem)
    cp_x.wait()
    cp_xs = pltpu.async_copy(xrows_v, xs_hbm.at[pos_v], sem2)
    cp_pos.wait()
    cp_xs.wait()

    @pl.when(wid == 0)
    def _():
        lane = lax.iota(jnp.int32, L)
        off_v[...] = excl
        offp1 = plsc.load_gather(off_v, [jnp.minimum(lane + 1, L - 1)])
        lo = excl // B
        nonempty = (offp1 > excl) & (lane < E)
        hi = jnp.where(nonempty, (jnp.maximum(offp1, 1) - 1) // B, lo)
        nblk = jnp.where(nonempty, hi - lo + 1, 0)
        winc = plsc.cumsum(nblk)
        wstart = winc - nblk
        lo_v[...] = lo
        ws_v[...] = wstart
        cnt_v = out16_v
        cnt_v[...] = zero
        plsc.addupdate_scatter(cnt_v, [wstart], ones, mask=lane < E)
        acc = plsc.cumsum(cnt_v[...])
        ew_raw = acc - 1
        lo_g = plsc.load_gather(lo_v, [ew_raw])
        ws_g = plsc.load_gather(ws_v, [ew_raw])
        nb_raw = lo_g + lane - ws_g
        ew_v[...] = ew_raw
        nb_v[...] = nb_raw
        nwork = jnp.max(winc)
        valid = lane < nwork
        last_v = lane * 0 + jnp.maximum(nwork - 1, 0)
        ew_last = plsc.load_gather(ew_v, [last_v])
        nb_last = plsc.load_gather(nb_v, [last_v])
        ew = jnp.where(valid, ew_raw, ew_last)
        nb = jnp.where(valid, nb_raw, nb_last)
        st = plsc.load_gather(off_v, [ew])
        en = plsc.load_gather(off_v, [ew + 1])
        out16_v[...] = ew
        pltpu.sync_copy(out16_v, ew_hbm)
        out16_v[...] = nb
        pltpu.sync_copy(out16_v, nb_hbm)
        out16_v[...] = valid.astype(jnp.int32)
        pltpu.sync_copy(out16_v, val_hbm)
        out16_v[...] = st
        pltpu.sync_copy(out16_v, st_hbm)
        out16_v[...] = en
        pltpu.sync_copy(out16_v, en_hbm)


def _unsort_body(zs_hbm, pos_hbm, out_hbm, pos_v, rows_v,
                 sem_a, sem_b, sem_c, sem_d):
    wid = _sc_wid()
    base = wid * CHUNK
    hh = CHUNK // 2
    pltpu.sync_copy(pos_hbm.at[pl.ds(base, CHUNK)], pos_v)
    g0 = pltpu.async_copy(zs_hbm.at[pos_v.at[pl.ds(0, hh)]],
                          rows_v.at[pl.ds(0, hh)], sem_a)
    g1 = pltpu.async_copy(zs_hbm.at[pos_v.at[pl.ds(hh, hh)]],
                          rows_v.at[pl.ds(hh, hh)], sem_b)
    g0.wait()
    s0 = pltpu.async_copy(rows_v.at[pl.ds(0, hh)],
                          out_hbm.at[pl.ds(base, hh)], sem_c)
    g1.wait()
    s1 = pltpu.async_copy(rows_v.at[pl.ds(hh, hh)],
                          out_hbm.at[pl.ds(base + hh, hh)], sem_d)
    s0.wait()
    s1.wait()


def _sc_mesh():
    return plsc.VectorSubcoreMesh(core_axis_name="c", subcore_axis_name="s",
                                  num_cores=NC, num_subcores=NS)

_I16 = jax.ShapeDtypeStruct((L,), jnp.int32)


@jax.jit
def _routing_sc(at, x):
    return pl.kernel(
        _routing_body,
        out_type=(
            jax.ShapeDtypeStruct((N,), jnp.int32),
            jax.ShapeDtypeStruct((N, D), jnp.float32),
            _I16, _I16, _I16, _I16, _I16,
        ),
        mesh=_sc_mesh(),
        compiler_params=pltpu.CompilerParams(needs_layout_passes=False),
        scratch_types=[
            pltpu.VMEM((N,), jnp.int32),
            pltpu.VMEM((CHUNK,), jnp.int32),
            pltpu.VMEM((CHUNK, D), jnp.float32),
            pltpu.VMEM((CHUNK,), jnp.int32),
        ] + [pltpu.VMEM((L,), jnp.int32)] * 9 + [
            pltpu.SemaphoreType.DMA,
            pltpu.SemaphoreType.DMA,
        ],
    )(at, x)


@jax.jit
def _unsort_sc(zs, pos):
    return pl.kernel(
        _unsort_body,
        out_type=jax.ShapeDtypeStruct((N, D), jnp.float32),
        mesh=_sc_mesh(),
        compiler_params=pltpu.CompilerParams(needs_layout_passes=False),
        scratch_types=[
            pltpu.VMEM((CHUNK,), jnp.int32),
            pltpu.VMEM((CHUNK, D), jnp.float32),
            pltpu.SemaphoreType.DMA,
            pltpu.SemaphoreType.DMA,
            pltpu.SemaphoreType.DMA,
            pltpu.SemaphoreType.DMA,
        ],
    )(zs, pos)


def _mlp_body(ew_ref, nb_ref, val_ref, st_ref, en_ref,
              x_ref, W1_ref, b1_ref, W2_ref, b2_ref, g_ref, out_ref):
    w = pl.program_id(0)

    @pl.when(val_ref[w] == 1)
    def _():
        blk_lo = nb_ref[w] * B
        start = st_ref[w]
        end = en_ref[w]
        x = x_ref[...]
        p = jax.lax.dot_general(
            x, W1_ref[0], (((1,), (1,)), ((), ())),
            preferred_element_type=jnp.float32)
        p = p + b1_ref[0]
        proj = p[:, :H]
        gate = p[:, H:]
        h = proj * (gate * jax.lax.logistic(gate))
        y = jax.lax.dot_general(
            h, W2_ref[0], (((1,), (1,)), ((), ())),
            preferred_element_type=jnp.float32)
        y = y + b2_ref[0]
        z = x + y
        ms = jnp.mean(z * z, axis=-1, keepdims=True)
        z = z * jax.lax.rsqrt(ms + EPS) * g_ref[0]
        gid = blk_lo + jax.lax.broadcasted_iota(jnp.int32, (B, 1), 0)
        mask = (gid >= start) & (gid < end)
        out_ref[...] = jnp.where(mask, z, out_ref[...])


@functools.partial(jax.jit, static_argnames=("interpret",))
def _mlp_sorted(ew, nb, val, st, en, x_sorted, W1, b1, W2, b2, g,
                interpret=False):
    grid_spec = pltpu.PrefetchScalarGridSpec(
        num_scalar_prefetch=5,
        grid=(NWORK,),
        in_specs=[
            pl.BlockSpec((B, D), lambda w, ew, nb, val, st, en: (nb[w], 0)),
            pl.BlockSpec((1, 2 * H, D),
                         lambda w, ew, nb, val, st, en: (ew[w], 0, 0)),
            pl.BlockSpec((1, 1, 2 * H),
                         lambda w, ew, nb, val, st, en: (ew[w], 0, 0)),
            pl.BlockSpec((1, D, H),
                         lambda w, ew, nb, val, st, en: (ew[w], 0, 0)),
            pl.BlockSpec((1, 1, D),
                         lambda w, ew, nb, val, st, en: (ew[w], 0, 0)),
            pl.BlockSpec((1, D), lambda w, ew, nb, val, st, en: (0, 0)),
        ],
        out_specs=pl.BlockSpec((B, D), lambda w, ew, nb, val, st, en: (nb[w], 0)),
    )
    return pl.pallas_call(
        _mlp_body,
        grid_spec=grid_spec,
        out_shape=jax.ShapeDtypeStruct((N, D), jnp.float32),
        compiler_params=pltpu.CompilerParams(
            dimension_semantics=("arbitrary",)),
        interpret=interpret,
    )(ew, nb, val, st, en, x_sorted,
      W1, b1.reshape(E, 1, 2 * H), W2, b2.reshape(E, 1, D),
      g.reshape(1, D))


def kernel(x, action_type, W1, b1, W2, b2, g):
    at = action_type.astype(jnp.int32)
    pos, x_sorted, ew, nb, val, st, en = _routing_sc(at, x)
    z_sorted = _mlp_sorted(ew, nb, val, st, en, x_sorted, W1, b1, W2, b2, g)
    return _unsort_sc(z_sorted, pos)
